# baseline (speedup 1.0000x reference)
import functools

import jax
import jax.numpy as jnp
from jax import lax
from jax.experimental import pallas as pl
from jax.experimental.pallas import tpu as pltpu
from jax.experimental.pallas import tpu_sc as plsc

B = 16384
D = 32
PACK = 4
VROWS = 100001
_TCBLK = 512
_TCGRID = (VROWS + _TCBLK - 1) // _TCBLK
PROWS = 28672

_info = plsc.get_sparse_core_info()
_NC = _info.num_cores
_NS = _info.num_subcores
_NW = _NC * _NS
_BPW = B // _NW
_CHUNK = 64
_NCHUNK = _BPW // _CHUNK

_mesh = plsc.VectorSubcoreMesh(core_axis_name="c", subcore_axis_name="s")


@functools.partial(
    pl.kernel,
    mesh=_mesh,
    out_type=jax.ShapeDtypeStruct((D, B), jnp.float32),
    scratch_types=[
        pltpu.VMEM((_BPW,), jnp.int32),
        pltpu.VMEM((_BPW,), jnp.int32),
        pltpu.VMEM((_BPW, 128), jnp.float32),
        pltpu.VMEM((D, _BPW), jnp.float32),
        pltpu.SemaphoreType.DMA,
    ],
    compiler_params=pltpu.CompilerParams(needs_layout_passes=False),
)
def _gather_kernel(tp_hbm, idx4_hbm, cofs_hbm, outT_hbm, idx4_v, cofs_v,
                   rows_v, outT_v, sem):
    wid = lax.axis_index("s") * _NC + lax.axis_index("c")
    base = wid * _BPW
    pltpu.sync_copy(idx4_hbm.at[pl.ds(base, _BPW)], idx4_v)
    pltpu.sync_copy(cofs_hbm.at[pl.ds(base, _BPW)], cofs_v)
    iota = lax.iota(jnp.int32, 16)

    copies = [
        pltpu.async_copy(
            tp_hbm.at[idx4_v.at[pl.ds(j * _CHUNK, _CHUNK)]],
            rows_v.at[pl.ds(j * _CHUNK, _CHUNK)],
            sem,
        )
        for j in range(_NCHUNK)
    ]

    _GPC = _CHUNK // 16

    for k in range(_NCHUNK):
        copies[k].wait()

        @plsc.parallel_loop(k * _GPC, (k + 1) * _GPC, unroll=4)
        def body(g):
            rid = g * 16 + iota
            cof = cofs_v[pl.ds(g * 16, 16)]
            for j in range(D):
                v = plsc.load_gather(rows_v, [rid, cof + j])
                outT_v[j, pl.ds(g * 16, 16)] = v
    pltpu.sync_copy(outT_v, outT_hbm.at[:, pl.ds(base, _BPW)])


_NG = 32
_TCGRID2 = (VROWS + _NG * _TCBLK - 1) // (_NG * _TCBLK)


def _pack_body(tT_ref, out_ref):
    t = tT_ref[...]
    for g in range(_NG):
        tg = t[:, g * 512:(g + 1) * 512]
        s = tg.reshape(D, PACK, 128).swapaxes(0, 1).reshape(128, 128)
        out_ref[g * 128:(g + 1) * 128, :] = jnp.transpose(s)


_pack = pl.pallas_call(
    _pack_body,
    grid=(_TCGRID2,),
    in_specs=[pl.BlockSpec((D, _NG * _TCBLK), lambda c: (0, c))],
    out_specs=pl.BlockSpec((_NG * 128, PACK * D), lambda c: (c, 0)),
    out_shape=jax.ShapeDtypeStruct((PROWS, PACK * D), jnp.float32),
)


def kernel(broadcaster, table):
    idx = broadcaster.astype(jnp.int32)
    idx4 = jnp.bitwise_or(
        jnp.left_shift(jnp.right_shift(idx, 9), 7),
        jnp.bitwise_and(idx, 127))
    cofs = jnp.left_shift(
        jnp.bitwise_and(jnp.right_shift(idx, 7), 3), 5)
    tp = _pack(table.T)
    outT = _gather_kernel(tp, idx4, cofs)
    return outT.T
